# baseline (speedup 1.0000x reference)
import jax
import jax.numpy as jnp
from jax.experimental import pallas as pl
from jax.experimental.pallas import tpu as pltpu

IN_N = 4096
OUT_N = 1024
DELAYS = 16
V_TH = 1.0
TAU = 5.0

BJ = 32
HALF = OUT_N // 2
NSTEP = HALF // BJ


def _body(t_ref, x_ref, d_ref, s0_ref, s1_ref, w0_ref, w1_ref,
          out0_ref, out1_ref, o_ref):
    tval = t_ref[0, 0]

    @pl.when(pl.program_id(0) == 0)
    def _():
        xx = x_ref[...]
        tt = tval - xx - d_ref[...]
        mask = jnp.logical_and(xx != -1.0, tt >= 0.0)
        o_ref[...] = jnp.where(mask, tt * jnp.exp(1.0 - tt / TAU) / TAU, 0.0)

    o = o_ref[...][None]
    for w_ref, s_ref, out_ref in ((w0_ref, s0_ref, out0_ref),
                                  (w1_ref, s1_ref, out1_ref)):
        v = jnp.sum(w_ref[...] * o, axis=(1, 2))
        s_old = s_ref[...]
        fire = jnp.logical_and(s_old < 0.0, v[:, None] >= V_TH)
        out_ref[...] = jnp.where(fire, tval, s_old)


def kernel(t, x, w, d, s):
    wt = jnp.transpose(w, (0, 2, 1))
    x2 = x.reshape(1, IN_N)
    d2 = d.reshape(DELAYS, 1)
    t2 = jnp.asarray(t, jnp.float32).reshape(1, 1)
    s2 = s.reshape(OUT_N, 1)
    nhb = HALF // BJ

    out0, out1 = pl.pallas_call(
        _body,
        grid=(NSTEP,),
        in_specs=[
            pl.BlockSpec((1, 1), lambda c: (0, 0)),
            pl.BlockSpec((1, IN_N), lambda c: (0, 0)),
            pl.BlockSpec((DELAYS, 1), lambda c: (0, 0)),
            pl.BlockSpec((BJ, 1), lambda c: (c, 0)),
            pl.BlockSpec((BJ, 1), lambda c: (c + nhb, 0)),
            pl.BlockSpec((BJ, DELAYS, IN_N), lambda c: (c, 0, 0)),
            pl.BlockSpec((BJ, DELAYS, IN_N), lambda c: (c + nhb, 0, 0)),
        ],
        out_specs=[
            pl.BlockSpec((BJ, 1), lambda c: (c, 0)),
            pl.BlockSpec((BJ, 1), lambda c: (c, 0)),
        ],
        out_shape=[
            jax.ShapeDtypeStruct((HALF, 1), jnp.float32),
            jax.ShapeDtypeStruct((HALF, 1), jnp.float32),
        ],
        scratch_shapes=[pltpu.VMEM((DELAYS, IN_N), jnp.float32)],
    )(t2, x2, d2, s2, s2, wt, wt)

    return jnp.concatenate([out0.reshape(HALF), out1.reshape(HALF)])


# speedup vs baseline: 1.0354x; 1.0354x over previous
import jax
import jax.numpy as jnp
from jax.experimental import pallas as pl
from jax.experimental.pallas import tpu as pltpu

IN_N = 4096
OUT_N = 1024
DELAYS = 16
V_TH = 1.0
TAU = 5.0

BJ = 32
NSTEP = OUT_N // BJ


def _body(t_ref, x_ref, d_ref, s_ref, w_ref, out_ref, o_ref):
    tval = t_ref[0, 0]

    @pl.when(pl.program_id(0) == 0)
    def _():
        xx = x_ref[...]
        tt = tval - xx - d_ref[...]
        mask = jnp.logical_and(xx != -1.0, tt >= 0.0)
        o_ref[...] = jnp.where(mask, tt * jnp.exp(1.0 - tt / TAU) / TAU, 0.0)

    prod = w_ref[...] * o_ref[...][None]
    v = jnp.sum(prod, axis=(1, 2))
    s_old = s_ref[...]
    fire = jnp.logical_and(s_old < 0.0, v[:, None] >= V_TH)
    out_ref[...] = jnp.where(fire, tval, s_old)


def kernel(t, x, w, d, s):
    wt = jnp.transpose(w, (0, 2, 1))
    x2 = x.reshape(1, IN_N)
    d2 = d.reshape(DELAYS, 1)
    t2 = jnp.asarray(t, jnp.float32).reshape(1, 1)
    s2 = s.reshape(OUT_N, 1)

    out = pl.pallas_call(
        _body,
        grid=(NSTEP,),
        in_specs=[
            pl.BlockSpec((1, 1), lambda c: (0, 0)),
            pl.BlockSpec((1, IN_N), lambda c: (0, 0)),
            pl.BlockSpec((DELAYS, 1), lambda c: (0, 0)),
            pl.BlockSpec((BJ, 1), lambda c: (c, 0)),
            pl.BlockSpec((BJ, DELAYS, IN_N), lambda c: (c, 0, 0)),
        ],
        out_specs=pl.BlockSpec((BJ, 1), lambda c: (c, 0)),
        out_shape=jax.ShapeDtypeStruct((OUT_N, 1), jnp.float32),
        scratch_shapes=[pltpu.VMEM((DELAYS, IN_N), jnp.float32)],
    )(t2, x2, d2, s2, wt)

    return out.reshape(OUT_N)
